# baseline (speedup 1.0000x reference)
import functools

import jax
import jax.numpy as jnp
from jax import lax
from jax.experimental import pallas as pl
from jax.experimental.pallas import tpu as pltpu
from jax.experimental.pallas import tpu_sc as plsc

N = 10000
E = 320000
D = 128
DH = D // 2

NC = 2
NS = 16
CHUNK = 128
NCHUNK = 157
EPAD = NS * NCHUNK * CHUNK

RPS = 632
RPS_LAST = N - 15 * RPS


def _segsum_sc(xl, xr, src, dst):
    mesh = plsc.VectorSubcoreMesh(core_axis_name="c", subcore_axis_name="s")

    @functools.partial(
        pl.kernel,
        mesh=mesh,
        compiler_params=pltpu.CompilerParams(use_tc_tiling_on_sc=False),
        out_type=jax.ShapeDtypeStruct((2 * N, DH), jnp.float32),
        scratch_types=[
            pltpu.VMEM((NCHUNK, CHUNK), jnp.int32),
            pltpu.VMEM((NCHUNK, CHUNK), jnp.int32),
            pltpu.VMEM((3, CHUNK, DH), jnp.float32),
            pltpu.VMEM_SHARED((N, DH), jnp.float32),
            pltpu.SemaphoreType.DMA,
            pltpu.SemaphoreType.DMA,
        ],
    )
    def k(xl_hbm, xr_hbm, src_hbm, dst_hbm, out_hbm,
          src_v, dst_v, rows, acc, sem_g, sem_s):
        cid = lax.axis_index("c")
        sid = lax.axis_index("s")

        zeros16 = jnp.zeros((16,), jnp.float32)

        def zero_body(t, _):
            rows[0, t // (DH // 16), pl.ds((t % (DH // 16)) * 16, 16)] = zeros16
            return _

        lax.fori_loop(0, CHUNK * (DH // 16), zero_body, None)
        row0 = pl.multiple_of(sid * RPS, 8)

        def zero_acc(base, total):
            for off in range(0, total, CHUNK):
                size = min(CHUNK, total - off)
                pltpu.sync_copy(rows.at[0, pl.ds(0, size)],
                                acc.at[pl.ds(base + off, size)])

        @pl.when(sid < NS - 1)
        def _():
            zero_acc(row0, RPS)

        @pl.when(sid == NS - 1)
        def _():
            zero_acc((NS - 1) * RPS, RPS_LAST)

        pltpu.sync_copy(src_hbm.at[sid], src_v)
        pltpu.sync_copy(dst_hbm.at[sid], dst_v)

        def gather_start(j, slot):
            @pl.when(cid == 0)
            def _():
                pltpu.async_copy(xl_hbm.at[src_v.at[j]], rows.at[slot], sem_g)

            @pl.when(cid == 1)
            def _():
                pltpu.async_copy(xr_hbm.at[src_v.at[j]], rows.at[slot], sem_g)

        def gather_wait():
            pltpu.make_async_copy(xl_hbm.at[src_v.at[0]], rows.at[0],
                                  sem_g).wait()

        def scat_start(j, slot):
            pltpu.async_copy(rows.at[slot], acc.at[dst_v.at[j]], sem_s,
                             add=True)

        def scat_wait():
            pltpu.make_async_copy(rows.at[0], acc.at[dst_v.at[0]],
                                  sem_s).wait()

        gather_start(0, 0)

        plsc.subcore_barrier()

        def body(i, _):
            c3 = lax.rem(i, 3)
            gather_wait()

            @pl.when(i >= 2)
            def _():
                scat_wait()

            @pl.when(i + 1 < NCHUNK)
            def _():
                gather_start(i + 1, lax.rem(i + 1, 3))

            scat_start(i, c3)
            return _

        lax.fori_loop(0, NCHUNK, body, None)
        scat_wait()
        scat_wait()

        plsc.subcore_barrier()

        out0 = pl.multiple_of(cid * N + sid * RPS, 8)

        @pl.when(sid < NS - 1)
        def _():
            pltpu.sync_copy(acc.at[pl.ds(row0, RPS)],
                            out_hbm.at[pl.ds(out0, RPS)])

        @pl.when(sid == NS - 1)
        def _():
            pltpu.sync_copy(
                acc.at[pl.ds((NS - 1) * RPS, RPS_LAST)],
                out_hbm.at[pl.ds(cid * N + (NS - 1) * RPS, RPS_LAST)],
            )

    return k(xl, xr, src, dst)


def _mm_kernel(pl_ref, pr_ref, wl_ref, wr_ref, b_ref, o_ref):
    y = jnp.dot(pl_ref[...], wl_ref[...], preferred_element_type=jnp.float32,
                precision=jax.lax.Precision.HIGHEST)
    y += jnp.dot(pr_ref[...], wr_ref[...], preferred_element_type=jnp.float32,
                 precision=jax.lax.Precision.HIGHEST)
    o_ref[...] = jnp.maximum(y + b_ref[...], 0.0)


def _finish_tc(partials, W, b2):
    blk = 1000
    nblk = N // blk
    return pl.pallas_call(
        _mm_kernel,
        grid=(nblk,),
        in_specs=[
            pl.BlockSpec((blk, DH), lambda i: (i, 0)),
            pl.BlockSpec((blk, DH), lambda i: (i + nblk, 0)),
            pl.BlockSpec((DH, D), lambda i: (0, 0)),
            pl.BlockSpec((DH, D), lambda i: (1, 0)),
            pl.BlockSpec((1, D), lambda i: (0, 0)),
        ],
        out_specs=pl.BlockSpec((blk, D), lambda i: (i, 0)),
        out_shape=jax.ShapeDtypeStruct((N, D), jnp.float32),
    )(partials, partials, W, W, b2)


def kernel(x, edge_index, W, b):
    ei = edge_index.astype(jnp.int32)
    src = jnp.full((EPAD,), N, jnp.int32).at[:E].set(ei[0])
    dst = jnp.zeros((EPAD,), jnp.int32).at[:E].set(ei[1])
    zpad = jnp.zeros((8, DH), jnp.float32)
    xl = jnp.concatenate([x[:, :DH], zpad], axis=0)
    xr = jnp.concatenate([x[:, DH:], zpad], axis=0)
    partials = _segsum_sc(xl, xr,
                          src.reshape(NS, NCHUNK, CHUNK),
                          dst.reshape(NS, NCHUNK, CHUNK))
    return _finish_tc(partials, W, b.reshape(1, D))


# speedup vs baseline: 1.2943x; 1.2943x over previous
import functools

import jax
import jax.numpy as jnp
from jax import lax
from jax.experimental import pallas as pl
from jax.experimental.pallas import tpu as pltpu
from jax.experimental.pallas import tpu_sc as plsc

N = 10000
E = 320000
D = 128
DH = D // 2

NC = 2
NS = 16
CHUNK = 128
NCHUNK = 157
EPAD = NS * NCHUNK * CHUNK

NPAD = N + 8

RPS = 632
RPS_LAST = N - 15 * RPS
RPS_LAST_X = NPAD - 15 * RPS


def _segsum_sc(xl, xr, idx):
    mesh = plsc.VectorSubcoreMesh(core_axis_name="c", subcore_axis_name="s")

    @functools.partial(
        pl.kernel,
        mesh=mesh,
        compiler_params=pltpu.CompilerParams(use_tc_tiling_on_sc=False),
        out_type=jax.ShapeDtypeStruct((2 * N, DH), jnp.float32),
        scratch_types=[
            pltpu.VMEM((4, 2, CHUNK), jnp.int32),
            pltpu.VMEM((3, CHUNK, DH), jnp.float32),
            pltpu.VMEM_SHARED((NPAD, DH), jnp.float32),
            pltpu.VMEM_SHARED((N, DH), jnp.float32),
            pltpu.SemaphoreType.DMA,
            pltpu.SemaphoreType.DMA,
            pltpu.SemaphoreType.DMA,
        ],
    )
    def k(xl_hbm, xr_hbm, idx_hbm, out_hbm,
          ibuf, rows, xspm, acc, sem_i, sem_g, sem_s):
        cid = lax.axis_index("c")
        sid = lax.axis_index("s")

        zeros16 = jnp.zeros((16,), jnp.float32)

        def zero_body(t, _):
            rows[0, t // (DH // 16), pl.ds((t % (DH // 16)) * 16, 16)] = zeros16
            return _

        lax.fori_loop(0, CHUNK * (DH // 16), zero_body, None)
        row0 = pl.multiple_of(sid * RPS, 8)

        def zero_acc(base, total):
            for off in range(0, total, CHUNK):
                size = min(CHUNK, total - off)
                pltpu.sync_copy(rows.at[0, pl.ds(0, size)],
                                acc.at[pl.ds(base + off, size)])

        @pl.when(sid < NS - 1)
        def _():
            zero_acc(row0, RPS)

        @pl.when(sid == NS - 1)
        def _():
            zero_acc((NS - 1) * RPS, RPS_LAST)

        @pl.when(cid == 0)
        def _():
            @pl.when(sid < NS - 1)
            def _():
                pltpu.sync_copy(xl_hbm.at[pl.ds(row0, RPS)],
                                xspm.at[pl.ds(row0, RPS)])

            @pl.when(sid == NS - 1)
            def _():
                pltpu.sync_copy(xl_hbm.at[pl.ds((NS - 1) * RPS, RPS_LAST_X)],
                                xspm.at[pl.ds((NS - 1) * RPS, RPS_LAST_X)])

        @pl.when(cid == 1)
        def _():
            @pl.when(sid < NS - 1)
            def _():
                pltpu.sync_copy(xr_hbm.at[pl.ds(row0, RPS)],
                                xspm.at[pl.ds(row0, RPS)])

            @pl.when(sid == NS - 1)
            def _():
                pltpu.sync_copy(xr_hbm.at[pl.ds((NS - 1) * RPS, RPS_LAST_X)],
                                xspm.at[pl.ds((NS - 1) * RPS, RPS_LAST_X)])

        def idx_start(j, slot):
            pltpu.async_copy(idx_hbm.at[sid, j], ibuf.at[slot], sem_i)

        def idx_wait():
            pltpu.make_async_copy(idx_hbm.at[sid, 0], ibuf.at[0],
                                  sem_i).wait()

        def gather_start(slot4, slot3):
            pltpu.async_copy(xspm.at[ibuf.at[slot4, 0]], rows.at[slot3],
                             sem_g)

        def gather_wait():
            pltpu.make_async_copy(xspm.at[ibuf.at[0, 0]], rows.at[0],
                                  sem_g).wait()

        def scat_start(slot4, slot3):
            pltpu.async_copy(rows.at[slot3], acc.at[ibuf.at[slot4, 1]],
                             sem_s, add=True)

        def scat_wait():
            pltpu.make_async_copy(rows.at[0], acc.at[ibuf.at[0, 1]],
                                  sem_s).wait()

        pltpu.sync_copy(idx_hbm.at[sid, 0], ibuf.at[0])
        plsc.subcore_barrier()
        gather_start(0, 0)
        idx_start(1, 1)

        def body(i, _):
            c4 = lax.rem(i, 4)
            c3 = lax.rem(i, 3)
            gather_wait()

            @pl.when(i >= 2)
            def _():
                scat_wait()

            scat_start(c4, c3)

            @pl.when(i + 1 < NCHUNK)
            def _():
                idx_wait()
                gather_start(lax.rem(i + 1, 4), lax.rem(i + 1, 3))

            @pl.when(i + 2 < NCHUNK)
            def _():
                idx_start(i + 2, lax.rem(i + 2, 4))

            return _

        lax.fori_loop(0, NCHUNK, body, None)
        scat_wait()
        scat_wait()

        plsc.subcore_barrier()

        out0 = pl.multiple_of(cid * N + sid * RPS, 8)

        @pl.when(sid < NS - 1)
        def _():
            pltpu.sync_copy(acc.at[pl.ds(row0, RPS)],
                            out_hbm.at[pl.ds(out0, RPS)])

        @pl.when(sid == NS - 1)
        def _():
            pltpu.sync_copy(
                acc.at[pl.ds((NS - 1) * RPS, RPS_LAST)],
                out_hbm.at[pl.ds(cid * N + (NS - 1) * RPS, RPS_LAST)],
            )

    return k(xl, xr, idx)


def _mm_kernel(pl_ref, pr_ref, wl_ref, wr_ref, b_ref, o_ref):
    y = jnp.dot(pl_ref[...], wl_ref[...], preferred_element_type=jnp.float32,
                precision=jax.lax.Precision.HIGHEST)
    y += jnp.dot(pr_ref[...], wr_ref[...], preferred_element_type=jnp.float32,
                 precision=jax.lax.Precision.HIGHEST)
    o_ref[...] = jnp.maximum(y + b_ref[...], 0.0)


def _finish_tc(partials, W, b2):
    blk = 1000
    nblk = N // blk
    return pl.pallas_call(
        _mm_kernel,
        grid=(nblk,),
        in_specs=[
            pl.BlockSpec((blk, DH), lambda i: (i, 0)),
            pl.BlockSpec((blk, DH), lambda i: (i + nblk, 0)),
            pl.BlockSpec((DH, D), lambda i: (0, 0)),
            pl.BlockSpec((DH, D), lambda i: (1, 0)),
            pl.BlockSpec((1, D), lambda i: (0, 0)),
        ],
        out_specs=pl.BlockSpec((blk, D), lambda i: (i, 0)),
        out_shape=jax.ShapeDtypeStruct((N, D), jnp.float32),
    )(partials, partials, W, W, b2)


def kernel(x, edge_index, W, b):
    ei = edge_index.astype(jnp.int32)
    src = jnp.full((EPAD,), N, jnp.int32).at[:E].set(ei[0])
    dst = jnp.zeros((EPAD,), jnp.int32).at[:E].set(ei[1])
    idx = jnp.stack(
        [src.reshape(NS, NCHUNK, CHUNK), dst.reshape(NS, NCHUNK, CHUNK)],
        axis=2)
    zpad = jnp.zeros((NPAD - N, DH), jnp.float32)
    xl = jnp.concatenate([x[:, :DH], zpad], axis=0)
    xr = jnp.concatenate([x[:, DH:], zpad], axis=0)
    partials = _segsum_sc(xl, xr, idx)
    return _finish_tc(partials, W, b.reshape(1, D))


# speedup vs baseline: 1.3892x; 1.0734x over previous
import functools

import jax
import jax.numpy as jnp
from jax import lax
from jax.experimental import pallas as pl
from jax.experimental.pallas import tpu as pltpu
from jax.experimental.pallas import tpu_sc as plsc

N = 10000
E = 320000
D = 128
DH = D // 2

NC = 2
NS = 16
EPS = E // NS
CHUNK = 128
NCHUNK = EPS // CHUNK
TAIL = EPS - NCHUNK * CHUNK

RPS = 632
RPS_LAST = N - 15 * RPS


def _segsum_sc(x, src, dst):
    mesh = plsc.VectorSubcoreMesh(core_axis_name="c", subcore_axis_name="s")

    @functools.partial(
        pl.kernel,
        mesh=mesh,
        compiler_params=pltpu.CompilerParams(use_tc_tiling_on_sc=False),
        out_type=jax.ShapeDtypeStruct((2 * N, DH), jnp.float32),
        scratch_types=[
            pltpu.VMEM((4, 2, CHUNK), jnp.int32),
            pltpu.VMEM((3, CHUNK, DH), jnp.float32),
            pltpu.VMEM_SHARED((N, DH), jnp.float32),
            pltpu.VMEM_SHARED((N + 8, DH), jnp.float32),
            pltpu.SemaphoreType.DMA,
            pltpu.SemaphoreType.DMA,
            pltpu.SemaphoreType.DMA,
        ],
    )
    def k(x_hbm, src_hbm, dst_hbm, out_hbm,
          ibuf, rows, xspm, acc, sem_i, sem_g, sem_s):
        cid = lax.axis_index("c")
        sid = lax.axis_index("s")
        ebase = sid * EPS

        zeros16 = jnp.zeros((16,), jnp.float32)

        def zero_body(t, _):
            rows[0, t // (DH // 16), pl.ds((t % (DH // 16)) * 16, 16)] = zeros16
            return _

        lax.fori_loop(0, CHUNK * (DH // 16), zero_body, None)
        row0 = pl.multiple_of(sid * RPS, 8)

        def zero_acc(base, total):
            for off in range(0, total, CHUNK):
                size = min(CHUNK, total - off)
                pltpu.sync_copy(rows.at[0, pl.ds(0, size)],
                                acc.at[pl.ds(base + off, size)])

        def stage_x(base, total):
            @pl.when(cid == 0)
            def _():
                pltpu.sync_copy(x_hbm.at[pl.ds(base, total), pl.ds(0, DH)],
                                xspm.at[pl.ds(base, total)])

            @pl.when(cid == 1)
            def _():
                pltpu.sync_copy(x_hbm.at[pl.ds(base, total), pl.ds(DH, DH)],
                                xspm.at[pl.ds(base, total)])

        @pl.when(sid < NS - 1)
        def _():
            zero_acc(row0, RPS)
            stage_x(row0, RPS)

        @pl.when(sid == NS - 1)
        def _():
            zero_acc((NS - 1) * RPS, RPS_LAST)
            stage_x((NS - 1) * RPS, RPS_LAST)

        def idx_start(j, slot):
            pltpu.async_copy(src_hbm.at[pl.ds(ebase + j * CHUNK, CHUNK)],
                             ibuf.at[slot, 0], sem_i)
            pltpu.async_copy(dst_hbm.at[pl.ds(ebase + j * CHUNK, CHUNK)],
                             ibuf.at[slot, 1], sem_i)

        def idx_wait():
            pltpu.make_async_copy(src_hbm.at[pl.ds(0, CHUNK)],
                                  ibuf.at[0, 0], sem_i).wait()
            pltpu.make_async_copy(dst_hbm.at[pl.ds(0, CHUNK)],
                                  ibuf.at[0, 1], sem_i).wait()

        def gather_start(slot4, slot3):
            pltpu.async_copy(xspm.at[ibuf.at[slot4, 0]], rows.at[slot3],
                             sem_g)

        def gather_wait():
            pltpu.make_async_copy(xspm.at[ibuf.at[0, 0]], rows.at[0],
                                  sem_g).wait()

        def scat_start(slot4, slot3):
            pltpu.async_copy(rows.at[slot3], acc.at[ibuf.at[slot4, 1]],
                             sem_s, add=True)

        def scat_wait():
            pltpu.make_async_copy(rows.at[0], acc.at[ibuf.at[0, 1]],
                                  sem_s).wait()

        pltpu.sync_copy(src_hbm.at[pl.ds(ebase, CHUNK)], ibuf.at[0, 0])
        pltpu.sync_copy(dst_hbm.at[pl.ds(ebase, CHUNK)], ibuf.at[0, 1])
        plsc.subcore_barrier()
        gather_start(0, 0)
        idx_start(1, 1)

        def body(i, _):
            c4 = lax.rem(i, 4)
            c3 = lax.rem(i, 3)
            gather_wait()

            @pl.when(i >= 2)
            def _():
                scat_wait()

            scat_start(c4, c3)

            @pl.when(i + 1 < NCHUNK)
            def _():
                idx_wait()
                gather_start(lax.rem(i + 1, 4), lax.rem(i + 1, 3))

            @pl.when(i + 2 < NCHUNK)
            def _():
                idx_start(i + 2, lax.rem(i + 2, 4))

            return _

        lax.fori_loop(0, NCHUNK, body, None)
        scat_wait()
        scat_wait()

        tbase = ebase + NCHUNK * CHUNK
        pltpu.sync_copy(src_hbm.at[pl.ds(tbase, TAIL)],
                        ibuf.at[0, 0, pl.ds(0, TAIL)])
        pltpu.sync_copy(dst_hbm.at[pl.ds(tbase, TAIL)],
                        ibuf.at[0, 1, pl.ds(0, TAIL)])
        for t in range((CHUNK - TAIL) // 16):
            ibuf[0, 0, pl.ds(TAIL + t * 16, 16)] = jnp.zeros((16,), jnp.int32)
            ibuf[0, 1, pl.ds(TAIL + t * 16, 16)] = jnp.full((16,), N,
                                                            jnp.int32)
        pltpu.async_copy(xspm.at[ibuf.at[0, 0]], rows.at[0], sem_g).wait()
        pltpu.sync_copy(rows.at[0], acc.at[ibuf.at[0, 1]], add=True)

        plsc.subcore_barrier()

        out0 = pl.multiple_of(cid * N + sid * RPS, 8)

        @pl.when(sid < NS - 1)
        def _():
            pltpu.sync_copy(acc.at[pl.ds(row0, RPS)],
                            out_hbm.at[pl.ds(out0, RPS)])

        @pl.when(sid == NS - 1)
        def _():
            pltpu.sync_copy(
                acc.at[pl.ds((NS - 1) * RPS, RPS_LAST)],
                out_hbm.at[pl.ds(cid * N + (NS - 1) * RPS, RPS_LAST)],
            )

    return k(x, src, dst)


def _mm_kernel(pl_ref, pr_ref, wl_ref, wr_ref, b_ref, o_ref):
    y = jnp.dot(pl_ref[...], wl_ref[...], preferred_element_type=jnp.float32,
                precision=jax.lax.Precision.HIGHEST)
    y += jnp.dot(pr_ref[...], wr_ref[...], preferred_element_type=jnp.float32,
                 precision=jax.lax.Precision.HIGHEST)
    o_ref[...] = jnp.maximum(y + b_ref[...], 0.0)


def _finish_tc(partials, W, b2):
    blk = 1000
    nblk = N // blk
    return pl.pallas_call(
        _mm_kernel,
        grid=(nblk,),
        in_specs=[
            pl.BlockSpec((blk, DH), lambda i: (i, 0)),
            pl.BlockSpec((blk, DH), lambda i: (i + nblk, 0)),
            pl.BlockSpec((DH, D), lambda i: (0, 0)),
            pl.BlockSpec((DH, D), lambda i: (1, 0)),
            pl.BlockSpec((1, D), lambda i: (0, 0)),
        ],
        out_specs=pl.BlockSpec((blk, D), lambda i: (i, 0)),
        out_shape=jax.ShapeDtypeStruct((N, D), jnp.float32),
    )(partials, partials, W, W, b2)


def kernel(x, edge_index, W, b):
    ei = edge_index.astype(jnp.int32)
    partials = _segsum_sc(x, ei[0], ei[1])
    return _finish_tc(partials, W, b.reshape(1, D))


# speedup vs baseline: 1.5340x; 1.1042x over previous
import functools

import jax
import jax.numpy as jnp
from jax import lax
from jax.experimental import pallas as pl
from jax.experimental.pallas import tpu as pltpu
from jax.experimental.pallas import tpu_sc as plsc

N = 10000
E = 320000
D = 128
DH = D // 2

NC = 2
NS = 16
EPS = E // NS
CHUNK = 128
NCHUNK = EPS // CHUNK
TAIL = EPS - NCHUNK * CHUNK

RPS = 632
RPS_LAST = N - 15 * RPS


def _segsum_sc(x, ei):
    mesh = plsc.VectorSubcoreMesh(core_axis_name="c", subcore_axis_name="s")

    @functools.partial(
        pl.kernel,
        mesh=mesh,
        compiler_params=pltpu.CompilerParams(use_tc_tiling_on_sc=False),
        out_type=jax.ShapeDtypeStruct((2 * N, DH), jnp.float32),
        scratch_types=[
            pltpu.VMEM((4, 2, CHUNK), jnp.int32),
            pltpu.VMEM((3, CHUNK, DH), jnp.float32),
            pltpu.VMEM_SHARED((N, DH), jnp.float32),
            pltpu.VMEM_SHARED((N + 8, DH), jnp.float32),
            pltpu.SemaphoreType.DMA,
            pltpu.SemaphoreType.DMA,
            pltpu.SemaphoreType.DMA,
        ],
    )
    def k(x_hbm, ei_hbm, out_hbm,
          ibuf, rows, xspm, acc, sem_i, sem_g, sem_s):
        cid = lax.axis_index("c")
        sid = lax.axis_index("s")
        ebase = sid * EPS

        zeros16 = jnp.zeros((16,), jnp.float32)

        def zero_body(t, _):
            rows[0, t // (DH // 16), pl.ds((t % (DH // 16)) * 16, 16)] = zeros16
            return _

        lax.fori_loop(0, CHUNK * (DH // 16), zero_body, None)
        row0 = pl.multiple_of(sid * RPS, 8)

        def zero_acc(base, total):
            for off in range(0, total, CHUNK):
                size = min(CHUNK, total - off)
                pltpu.sync_copy(rows.at[0, pl.ds(0, size)],
                                acc.at[pl.ds(base + off, size)])

        def stage_x(base, total):
            @pl.when(cid == 0)
            def _():
                pltpu.sync_copy(x_hbm.at[pl.ds(base, total), pl.ds(0, DH)],
                                xspm.at[pl.ds(base, total)])

            @pl.when(cid == 1)
            def _():
                pltpu.sync_copy(x_hbm.at[pl.ds(base, total), pl.ds(DH, DH)],
                                xspm.at[pl.ds(base, total)])

        @pl.when(sid < NS - 1)
        def _():
            zero_acc(row0, RPS)
            stage_x(row0, RPS)

        @pl.when(sid == NS - 1)
        def _():
            zero_acc((NS - 1) * RPS, RPS_LAST)
            stage_x((NS - 1) * RPS, RPS_LAST)

        def idx_start(j, slot):
            pltpu.async_copy(ei_hbm.at[0, pl.ds(ebase + j * CHUNK, CHUNK)],
                             ibuf.at[slot, 0], sem_i)
            pltpu.async_copy(ei_hbm.at[1, pl.ds(ebase + j * CHUNK, CHUNK)],
                             ibuf.at[slot, 1], sem_i)

        def idx_wait():
            pltpu.make_async_copy(ei_hbm.at[0, pl.ds(0, CHUNK)],
                                  ibuf.at[0, 0], sem_i).wait()
            pltpu.make_async_copy(ei_hbm.at[1, pl.ds(0, CHUNK)],
                                  ibuf.at[0, 1], sem_i).wait()

        def gather_start(slot4, slot3):
            pltpu.async_copy(xspm.at[ibuf.at[slot4, 0]], rows.at[slot3],
                             sem_g)

        def gather_wait():
            pltpu.make_async_copy(xspm.at[ibuf.at[0, 0]], rows.at[0],
                                  sem_g).wait()

        def scat_start(slot4, slot3):
            pltpu.async_copy(rows.at[slot3], acc.at[ibuf.at[slot4, 1]],
                             sem_s, add=True)

        def scat_wait():
            pltpu.make_async_copy(rows.at[0], acc.at[ibuf.at[0, 1]],
                                  sem_s).wait()

        pltpu.sync_copy(ei_hbm.at[0, pl.ds(ebase, CHUNK)], ibuf.at[0, 0])
        pltpu.sync_copy(ei_hbm.at[1, pl.ds(ebase, CHUNK)], ibuf.at[0, 1])
        plsc.subcore_barrier()
        gather_start(0, 0)
        idx_start(1, 1)

        def body(i, _):
            c4 = lax.rem(i, 4)
            c3 = lax.rem(i, 3)
            gather_wait()

            @pl.when(i >= 2)
            def _():
                scat_wait()

            scat_start(c4, c3)

            @pl.when(i + 1 < NCHUNK)
            def _():
                idx_wait()
                gather_start(lax.rem(i + 1, 4), lax.rem(i + 1, 3))

            @pl.when(i + 2 < NCHUNK)
            def _():
                idx_start(i + 2, lax.rem(i + 2, 4))

            return _

        lax.fori_loop(0, NCHUNK, body, None)
        scat_wait()
        scat_wait()

        tbase = ebase + NCHUNK * CHUNK
        pltpu.sync_copy(ei_hbm.at[0, pl.ds(tbase, TAIL)],
                        ibuf.at[0, 0, pl.ds(0, TAIL)])
        pltpu.sync_copy(ei_hbm.at[1, pl.ds(tbase, TAIL)],
                        ibuf.at[0, 1, pl.ds(0, TAIL)])
        for t in range((CHUNK - TAIL) // 16):
            ibuf[0, 0, pl.ds(TAIL + t * 16, 16)] = jnp.zeros((16,), jnp.int32)
            ibuf[0, 1, pl.ds(TAIL + t * 16, 16)] = jnp.full((16,), N,
                                                            jnp.int32)
        pltpu.async_copy(xspm.at[ibuf.at[0, 0]], rows.at[0], sem_g).wait()
        pltpu.sync_copy(rows.at[0], acc.at[ibuf.at[0, 1]], add=True)

        plsc.subcore_barrier()

        out0 = pl.multiple_of(cid * N + sid * RPS, 8)

        @pl.when(sid < NS - 1)
        def _():
            pltpu.sync_copy(acc.at[pl.ds(row0, RPS)],
                            out_hbm.at[pl.ds(out0, RPS)])

        @pl.when(sid == NS - 1)
        def _():
            pltpu.sync_copy(
                acc.at[pl.ds((NS - 1) * RPS, RPS_LAST)],
                out_hbm.at[pl.ds(cid * N + (NS - 1) * RPS, RPS_LAST)],
            )

    return k(x, ei)


def _mm_kernel(pl_ref, pr_ref, wl_ref, wr_ref, b_ref, o_ref):
    y = jnp.dot(pl_ref[...], wl_ref[...], preferred_element_type=jnp.float32,
                precision=jax.lax.Precision.HIGHEST)
    y += jnp.dot(pr_ref[...], wr_ref[...], preferred_element_type=jnp.float32,
                 precision=jax.lax.Precision.HIGHEST)
    o_ref[...] = jnp.maximum(y + b_ref[...], 0.0)


def _finish_tc(partials, W, b2):
    blk = 2000
    nblk = N // blk
    return pl.pallas_call(
        _mm_kernel,
        grid=(nblk,),
        in_specs=[
            pl.BlockSpec((blk, DH), lambda i: (i, 0)),
            pl.BlockSpec((blk, DH), lambda i: (i + nblk, 0)),
            pl.BlockSpec((DH, D), lambda i: (0, 0)),
            pl.BlockSpec((DH, D), lambda i: (1, 0)),
            pl.BlockSpec((1, D), lambda i: (0, 0)),
        ],
        out_specs=pl.BlockSpec((blk, D), lambda i: (i, 0)),
        out_shape=jax.ShapeDtypeStruct((N, D), jnp.float32),
    )(partials, partials, W, W, b2)


def kernel(x, edge_index, W, b):
    partials = _segsum_sc(x, edge_index.astype(jnp.int32))
    return _finish_tc(partials, W, b.reshape(1, D))
